# baseline (speedup 1.0000x reference)
import functools

import jax
import jax.numpy as jnp
from jax import lax
from jax.experimental import pallas as pl
from jax.experimental.pallas import tpu as pltpu
from jax.experimental.pallas import tpu_sc as plsc

_VOCAB = 1_000_000
_D = 64
_B = 16384
_NC = 2
_NS = 16
_L = 16
_NW = _NC * _NS
_BPW = _B // _NW
_G = _BPW // _L

_mesh = plsc.VectorSubcoreMesh(
    core_axis_name="c", subcore_axis_name="s", num_cores=_NC, num_subcores=_NS
)


@functools.partial(
    pl.kernel,
    out_type=jax.ShapeDtypeStruct((_B,), jnp.float32),
    mesh=_mesh,
    compiler_params=pltpu.CompilerParams(
        needs_layout_passes=False, use_tc_tiling_on_sc=False
    ),
    scratch_types=[
        pltpu.VMEM((_BPW,), jnp.int32),
        pltpu.VMEM((_BPW,), jnp.int32),
        pltpu.VMEM((_BPW, _D), jnp.float32),
        pltpu.VMEM((_BPW, _D), jnp.float32),
        pltpu.VMEM((_BPW,), jnp.float32),
        pltpu.VMEM((_BPW,), jnp.float32),
        pltpu.VMEM((_BPW,), jnp.float32),
        pltpu.SemaphoreType.DMA,
        pltpu.SemaphoreType.DMA,
        pltpu.SemaphoreType.DMA,
        pltpu.SemaphoreType.DMA,
    ],
)
def _glove_sc(left_hbm, right_hbm, lemb_hbm, lbias_hbm, remb_hbm, rbias_hbm,
              out_hbm, lidx, ridx, lrows, rrows, lb, rb, outv,
              sem0, sem1, sem2, sem3):
    wid = lax.axis_index("s") * _NC + lax.axis_index("c")
    base = wid * _BPW

    pltpu.sync_copy(left_hbm.at[pl.ds(base, _BPW)], lidx)
    pltpu.sync_copy(right_hbm.at[pl.ds(base, _BPW)], ridx)

    c0 = pltpu.async_copy(lemb_hbm.at[lidx], lrows, sem0)
    c1 = pltpu.async_copy(remb_hbm.at[ridx], rrows, sem1)
    c2 = pltpu.async_copy(lbias_hbm.at[lidx], lb, sem2)
    c3 = pltpu.async_copy(rbias_hbm.at[ridx], rb, sem3)
    c0.wait()
    c1.wait()
    c2.wait()
    c3.wait()

    lane = lax.iota(jnp.int32, _L)

    def group(g, carry):
        rows = jnp.full((_L,), g * _L, jnp.int32) + lane
        acc = plsc.load_gather(lb, [rows]) + plsc.load_gather(rb, [rows])
        for c in range(_D):
            col = jnp.full((_L,), c, jnp.int32)
            acc = acc + plsc.load_gather(lrows, [rows, col]) * plsc.load_gather(rrows, [rows, col])
        outv[pl.ds(pl.multiple_of(g * _L, _L), _L)] = acc
        return carry

    lax.fori_loop(0, _G, group, 0)

    pltpu.sync_copy(outv, out_hbm.at[pl.ds(base, _BPW)])


def kernel(left_id, right_id, l_emb, l_bias, r_emb, r_bias):
    return _glove_sc(
        left_id.astype(jnp.int32), right_id.astype(jnp.int32),
        l_emb, l_bias.reshape(_VOCAB), r_emb, r_bias.reshape(_VOCAB),
    )
